# Initial kernel scaffold; baseline (speedup 1.0000x reference)
import jax
import jax.numpy as jnp
from jax.experimental import pallas as pl


def kernel(drug_x, edge_index, batch, gexpr, W1, b1, g1, be1, W2, b2, g2, be2, Wc1, bc1, gc1, bec1, Wc2, bc2):
    raise NotImplementedError("write your pallas kernel here")


# speedup vs baseline: 7.4702x; 7.4702x over previous
import functools

import jax
import jax.numpy as jnp
from jax import lax
from jax.experimental import pallas as pl
from jax.experimental.pallas import tpu as pltpu
from jax.experimental.pallas import tpu_sc as plsc

N = 10000
E = 320000
B = 256
OUT = 128
NP = 10240
CHUNK = 128
NCORE = 2
NSUB = 16
NTILE = NCORE * NSUB
TILE_E = 10112
EPAD = NTILE * TILE_E
NCHUNK = TILE_E // CHUNK
ZROWS = NP // NSUB
DEGW = 128
EPS = 1e-5


def _deg_body(dst_hbm, zeros_hbm, ones_hbm, out_hbm, idxd, rows, acc):
    c = lax.axis_index("c")
    s = lax.axis_index("s")
    wid = c * NSUB + s
    pltpu.sync_copy(zeros_hbm, rows)
    for t in range(ZROWS // CHUNK):
        pltpu.sync_copy(rows, acc.at[pl.ds(s * ZROWS + t * CHUNK, CHUNK)])
    plsc.subcore_barrier()
    pltpu.sync_copy(ones_hbm, rows)

    def chunk(j, carry):
        off = wid * TILE_E + j * CHUNK
        pltpu.sync_copy(dst_hbm.at[pl.ds(off, CHUNK)], idxd)
        pltpu.sync_copy(rows, acc.at[idxd], add=True)
        return carry

    lax.fori_loop(0, NCHUNK, chunk, 0)
    plsc.subcore_barrier()
    pltpu.sync_copy(acc.at[pl.ds(s * ZROWS, ZROWS)],
                    out_hbm.at[pl.ds(c * NP + s * ZROWS, ZROWS)])


def _agg_body(vals_hbm, src_hbm, dst_hbm, zeros_hbm, out_hbm,
              idxs, idxd, rows, acc, sem):
    c = lax.axis_index("c")
    s = lax.axis_index("s")
    wid = c * NSUB + s
    pltpu.sync_copy(zeros_hbm, rows)
    for t in range(ZROWS // CHUNK):
        pltpu.sync_copy(rows, acc.at[pl.ds(s * ZROWS + t * CHUNK, CHUNK)])
    plsc.subcore_barrier()

    def chunk(j, carry):
        off = wid * TILE_E + j * CHUNK
        pltpu.sync_copy(src_hbm.at[pl.ds(off, CHUNK)], idxs)
        pltpu.sync_copy(dst_hbm.at[pl.ds(off, CHUNK)], idxd)
        pltpu.async_copy(vals_hbm.at[idxs], rows, sem).wait()
        pltpu.sync_copy(rows, acc.at[idxd], add=True)
        return carry

    lax.fori_loop(0, NCHUNK, chunk, 0)
    plsc.subcore_barrier()
    pltpu.sync_copy(acc.at[pl.ds(s * ZROWS, ZROWS)],
                    out_hbm.at[pl.ds(c * NP + s * ZROWS, ZROWS)])


@functools.cache
def _sc_kernels():
    mesh = plsc.VectorSubcoreMesh(core_axis_name="c", subcore_axis_name="s",
                                  num_cores=NCORE, num_subcores=NSUB)
    deg_k = pl.kernel(
        _deg_body,
        out_type=jax.ShapeDtypeStruct((NCORE * NP, DEGW), jnp.float32),
        mesh=mesh,
        scratch_types=[
            pltpu.VMEM((CHUNK,), jnp.int32),
            pltpu.VMEM((CHUNK, DEGW), jnp.float32),
            pltpu.VMEM_SHARED((NP, DEGW), jnp.float32),
        ],
    )
    agg_k = pl.kernel(
        _agg_body,
        out_type=jax.ShapeDtypeStruct((NCORE * NP, OUT), jnp.float32),
        mesh=mesh,
        scratch_types=[
            pltpu.VMEM((CHUNK,), jnp.int32),
            pltpu.VMEM((CHUNK,), jnp.int32),
            pltpu.VMEM((CHUNK, OUT), jnp.float32),
            pltpu.VMEM_SHARED((NP, OUT), jnp.float32),
            pltpu.SemaphoreType.DMA,
        ],
    )
    return deg_k, agg_k


def _tc1_body(degp, xp, w1, hs1, dinvb):
    deg = degp[0, :, 0:1] + degp[1, :, 0:1] + 1.0
    db = jnp.broadcast_to(lax.rsqrt(deg), (NP, OUT))
    dinvb[...] = db
    h = jnp.dot(xp[...], w1[...], preferred_element_type=jnp.float32)
    hs1[...] = h * db


_tc1 = pl.pallas_call(
    _tc1_body,
    out_shape=[
        jax.ShapeDtypeStruct((NP, OUT), jnp.float32),
        jax.ShapeDtypeStruct((NP, OUT), jnp.float32),
    ],
)


def _bn_masked(a):
    rid = lax.broadcasted_iota(jnp.int32, (NP, 1), 0)
    m = rid < N
    am = jnp.where(m, a, 0.0)
    mu = jnp.sum(am, axis=0, keepdims=True) / N
    d = jnp.where(m, a - mu, 0.0)
    var = jnp.sum(d * d, axis=0, keepdims=True) / N
    return mu, var


def _tc2_body(sp, hsp, dinvb, b, g, be, w, out):
    db = dinvb[...]
    z = db * (sp[0] + sp[1] + hsp[...]) + b[...]
    a = jnp.maximum(z, 0.0)
    mu, var = _bn_masked(a)
    y = (a - mu) * lax.rsqrt(var + EPS) * g[...] + be[...]
    out[...] = jnp.dot(y, w[...], preferred_element_type=jnp.float32) * db


_tc2 = pl.pallas_call(
    _tc2_body,
    out_shape=jax.ShapeDtypeStruct((NP, OUT), jnp.float32),
)


def _tc3_body(sp, hsp, dinvb, b, g, be, batchc, gx, wc1, bc1, gc1, bec1,
              wc2, bc2, xdrug, xcell):
    z = dinvb[...] * (sp[0] + sp[1] + hsp[...]) + b[...]
    a = jnp.maximum(z, 0.0)
    mu, var = _bn_masked(a)
    y = (a - mu) * lax.rsqrt(var + EPS) * g[...] + be[...]
    rid = lax.broadcasted_iota(jnp.int32, (NP, 1), 0)
    ym = jnp.where(rid < N, y, -jnp.inf)
    bc = batchc[...]

    def seg(bi, carry):
        vals = jnp.where(bc == bi, ym, -jnp.inf)
        xdrug[pl.ds(bi, 1), :] = jnp.max(vals, axis=0, keepdims=True)
        return carry

    lax.fori_loop(0, B, seg, 0)

    t = jnp.tanh(jnp.dot(gx[...], wc1[...],
                         preferred_element_type=jnp.float32) + bc1[...])
    cmu = jnp.mean(t, axis=0, keepdims=True)
    cvar = jnp.mean((t - cmu) ** 2, axis=0, keepdims=True)
    yc = (t - cmu) * lax.rsqrt(cvar + EPS) * gc1[...] + bec1[...]
    xcell[...] = jnp.maximum(
        jnp.dot(yc, wc2[...], preferred_element_type=jnp.float32) + bc2[...],
        0.0)


_tc3 = pl.pallas_call(
    _tc3_body,
    out_shape=[
        jax.ShapeDtypeStruct((B, OUT), jnp.float32),
        jax.ShapeDtypeStruct((B, OUT), jnp.float32),
    ],
)


def kernel(drug_x, edge_index, batch, gexpr, W1, b1, g1, be1, W2, b2, g2, be2,
           Wc1, bc1, gc1, bec1, Wc2, bc2):
    src = edge_index[0].astype(jnp.int32)
    dst = edge_index[1].astype(jnp.int32)
    pad = jnp.full((EPAD - E,), N, jnp.int32)
    srcp = jnp.concatenate([src, pad])
    dstp = jnp.concatenate([dst, pad])
    xp = jnp.pad(drug_x, ((0, NP - N), (0, 0)))
    zeros_f = jnp.zeros((CHUNK, OUT), jnp.float32)
    zeros_d = jnp.zeros((CHUNK, DEGW), jnp.float32)
    ones_d = jnp.ones((CHUNK, DEGW), jnp.float32)

    deg_k, agg_k = _sc_kernels()
    degp = deg_k(dstp, zeros_d, ones_d).reshape(NCORE, NP, DEGW)
    hs1, dinvb = _tc1(degp, xp, W1)
    s1 = agg_k(hs1, srcp, dstp, zeros_f).reshape(NCORE, NP, OUT)
    hs2 = _tc2(s1, hs1, dinvb, b1.reshape(1, OUT), g1.reshape(1, OUT),
               be1.reshape(1, OUT), W2)
    s2 = agg_k(hs2, srcp, dstp, zeros_f).reshape(NCORE, NP, OUT)

    batchc = jnp.pad(batch.astype(jnp.int32), (0, NP - N)).reshape(NP, 1)
    gxp = jnp.pad(gexpr, ((0, 0), (0, 1024 - gexpr.shape[1])))
    wc1p = jnp.pad(Wc1, ((0, 1024 - Wc1.shape[0]), (0, 0)))
    x_drug, x_cell = _tc3(
        s2, hs2, dinvb, b2.reshape(1, OUT), g2.reshape(1, OUT),
        be2.reshape(1, OUT), batchc, gxp, wc1p, bc1.reshape(1, OUT),
        gc1.reshape(1, OUT), bec1.reshape(1, OUT), Wc2, bc2.reshape(1, OUT))
    return (x_drug, x_cell)
